# baseline (speedup 1.0000x reference)
import jax
import jax.numpy as jnp
from jax.experimental import pallas as pl
from jax.experimental.pallas import tpu as pltpu
from jax.experimental.pallas import tpu_sc as plsc

B, D, T = 8, 1024, 512
CD, K = 64, 8192
KB = 2048
NKB = K // KB
RG = 8
NIDX = B * T
GWIN = 128
CDP = 2 * CD


def _search_body(z_ref, w_ref, bias_ref, cba_ref, cbb_ref,
                 lat_ref, gidx_ref, par_ref, idx_ref, cbn_ref, s_ref):
    b = pl.program_id(0)

    @pl.when(b == 0)
    def _():
        for i, cb_ref in enumerate((cba_ref, cbb_ref)):
            cb = cb_ref[...]
            n = jnp.sqrt(jnp.sum(cb * cb, axis=1, keepdims=True))
            cbn = cb / jnp.maximum(n, 1e-12)
            cbn_ref[i, :, 0:CD] = cbn
            cbn_ref[i, :, CD:CD + 1] = jnp.sum(cbn * cbn, axis=1,
                                               keepdims=True)

    z = z_ref[0]
    lat = jax.lax.dot_general(
        w_ref[...], z, (((1,), (0,)), ((), ())),
        preferred_element_type=jnp.float32) + bias_ref[:, 0:1]
    lat_ref[0] = lat

    idx2 = []
    for i in range(2):
        enc = lat[i * CD:(i + 1) * CD, :]
        n = jnp.sqrt(jnp.sum(enc * enc, axis=0, keepdims=True))
        encn = enc / jnp.maximum(n, 1e-12)
        rn2 = jnp.sum(encn * encn, axis=0, keepdims=True)
        rn2b = jnp.broadcast_to(rn2, (RG, T))
        rowoff = jax.lax.broadcasted_iota(jnp.int32, (RG, T), 0)

        def reduce_block(k, carry):
            sref = s_ref.at[k % 2]

            def grp(g, c):
                rv, ri = c
                s8 = sref[pl.ds(g * RG, RG), :]
                c8 = jnp.broadcast_to(
                    cbn_ref[i, pl.ds(k * KB + g * RG, RG), CD:CD + 1],
                    (RG, T))
                d = (rn2b - 2.0 * s8) + c8
                lt = d < rv
                rv = jnp.where(lt, d, rv)
                ri = jnp.where(lt, rowoff + (k * KB + g * RG), ri)
                return rv, ri

            return jax.lax.fori_loop(0, KB // RG, grp, carry, unroll=8)

        carry = (jnp.full((RG, T), jnp.inf, jnp.float32),
                 jnp.zeros((RG, T), jnp.int32))
        for k in range(NKB):
            s_ref[k % 2] = jax.lax.dot_general(
                cbn_ref[i, k * KB:(k + 1) * KB, 0:CD], encn,
                (((1,), (0,)), ((), ())),
                preferred_element_type=jnp.float32)
            if k >= 1:
                carry = reduce_block(k - 1, carry)
        carry = reduce_block(NKB - 1, carry)

        run_v, run_i = carry
        vmin = jnp.min(run_v, axis=0, keepdims=True)
        cand = jnp.where(run_v == vmin, run_i, K)
        idx2.append(jnp.min(cand, axis=0, keepdims=True))

    gidx_ref[0, 0:1, :] = idx2[0] // 2
    gidx_ref[0, 1:2, :] = idx2[1] // 2
    par_ref[0, 0:1, :] = idx2[0] % 2
    par_ref[0, 1:2, :] = idx2[1] % 2
    idx_ref[0] = idx2[0] * K + idx2[1]


def _out_body(lat_ref, qa_ref, qb_ref, par_ref, woa_ref, wob_ref,
              boa_ref, bob_ref, zq_ref, loss_ref):
    lat = lat_ref[0]
    loss = jnp.zeros((), jnp.float32)
    for i, (q_ref, w_ref, bo_ref) in enumerate(
            ((qa_ref, woa_ref, boa_ref), (qb_ref, wob_ref, bob_ref))):
        z_i = lat[i * CD:(i + 1) * CD, :]
        q = q_ref[0]
        pr = jnp.transpose(par_ref[0, i:i + 1, :], (1, 0))
        qsel = jnp.where(pr == 1, q[:, CD:2 * CD], q[:, 0:CD])
        qT = jnp.transpose(qsel, (1, 0))
        diff = z_i - qT
        loss = loss + jnp.sum(diff * diff) / float(CD * T)
        st = z_i + (qT - z_i)
        zq = jax.lax.dot_general(
            w_ref[...], st, (((1,), (0,)), ((), ())),
            preferred_element_type=jnp.float32) + bo_ref[:, 0:1]
        zq_ref[0, i * (D // 2):(i + 1) * (D // 2), :] = zq
    loss_ref[0, 0, :] = jnp.full((128,), loss, jnp.float32)


def _sc_gather(cba2, cbb2, ia, ib):
    mesh = plsc.VectorSubcoreMesh(core_axis_name="c", subcore_axis_name="s")
    out = jax.ShapeDtypeStruct((NIDX, CDP), jnp.float32)

    @pl.kernel(out_type=(out, out), mesh=mesh)
    def gk(cba_hbm, cbb_hbm, ia_hbm, ib_hbm, oa_hbm, ob_hbm):
        for cb_hbm, i_hbm, o_hbm in ((cba_hbm, ia_hbm, oa_hbm),
                                     (cbb_hbm, ib_hbm, ob_hbm)):
            def body(i_vmem, o_vmem, cb=cb_hbm):
                pltpu.sync_copy(cb.at[i_vmem.at[0]], o_vmem)

            pltpu.emit_pipeline(
                body,
                grid=(NIDX // GWIN,),
                in_specs=[pl.BlockSpec((1, GWIN), lambda i: (0, i))],
                out_specs=[pl.BlockSpec((GWIN, CDP), lambda i: (i, 0))],
                core_axis_name=("c", "s"),
                dimension_semantics=(pltpu.PARALLEL,),
            )(i_hbm, o_hbm)

    return gk(cba2, cbb2, ia, ib)


def kernel(z, W_in_a, b_in_a, W_in_b, b_in_b,
           W_out_a, b_out_a, W_out_b, b_out_b,
           codebook_a, codebook_b):
    f32 = jnp.float32
    w_stack = jnp.concatenate([W_in_a, W_in_b], axis=0)
    bias2d = jnp.broadcast_to(
        jnp.concatenate([b_in_a, b_in_b])[:, None], (2 * CD, 128))

    lat, gidx, par, idx3 = pl.pallas_call(
        _search_body,
        grid=(B,),
        in_specs=[
            pl.BlockSpec((1, D, T), lambda b: (b, 0, 0)),
            pl.BlockSpec((2 * CD, D), lambda b: (0, 0)),
            pl.BlockSpec((2 * CD, 128), lambda b: (0, 0)),
            pl.BlockSpec((K, CD), lambda b: (0, 0)),
            pl.BlockSpec((K, CD), lambda b: (0, 0)),
        ],
        out_specs=[
            pl.BlockSpec((1, 2 * CD, T), lambda b: (b, 0, 0)),
            pl.BlockSpec((1, 2, T), lambda b: (b, 0, 0)),
            pl.BlockSpec((1, 2, T), lambda b: (b, 0, 0)),
            pl.BlockSpec((1, 1, T), lambda b: (b, 0, 0)),
        ],
        out_shape=[
            jax.ShapeDtypeStruct((B, 2 * CD, T), f32),
            jax.ShapeDtypeStruct((B, 2, T), jnp.int32),
            jax.ShapeDtypeStruct((B, 2, T), jnp.int32),
            jax.ShapeDtypeStruct((B, 1, T), jnp.int32),
        ],
        scratch_shapes=[pltpu.VMEM((2, K, CD + 1), f32),
                        pltpu.VMEM((2, KB, T), f32)],
    )(z, w_stack, bias2d, codebook_a, codebook_b)

    indices = idx3[:, 0, :]
    gi = gidx.transpose(1, 0, 2).reshape(2, 1, NIDX)
    cba2 = codebook_a.reshape(K // 2, CDP)
    cbb2 = codebook_b.reshape(K // 2, CDP)

    qa, qb = _sc_gather(cba2, cbb2, gi[0], gi[1])
    qa = qa.reshape(B, T, CDP)
    qb = qb.reshape(B, T, CDP)

    bo_a = jnp.broadcast_to(b_out_a[:, None], (D // 2, 128))
    bo_b = jnp.broadcast_to(b_out_b[:, None], (D // 2, 128))

    zq, loss3 = pl.pallas_call(
        _out_body,
        grid=(B,),
        in_specs=[
            pl.BlockSpec((1, 2 * CD, T), lambda b: (b, 0, 0)),
            pl.BlockSpec((1, T, CDP), lambda b: (b, 0, 0)),
            pl.BlockSpec((1, T, CDP), lambda b: (b, 0, 0)),
            pl.BlockSpec((1, 2, T), lambda b: (b, 0, 0)),
            pl.BlockSpec((D // 2, CD), lambda b: (0, 0)),
            pl.BlockSpec((D // 2, CD), lambda b: (0, 0)),
            pl.BlockSpec((D // 2, 128), lambda b: (0, 0)),
            pl.BlockSpec((D // 2, 128), lambda b: (0, 0)),
        ],
        out_specs=[
            pl.BlockSpec((1, D, T), lambda b: (b, 0, 0)),
            pl.BlockSpec((1, 1, 128), lambda b: (b, 0, 0)),
        ],
        out_shape=[
            jax.ShapeDtypeStruct((B, D, T), f32),
            jax.ShapeDtypeStruct((B, 1, 128), f32),
        ],
    )(lat, qa, qb, par, W_out_a, W_out_b, bo_a, bo_b)

    loss = loss3[:, 0, 0]
    return (zq, loss, loss, indices, lat)
